# baseline (speedup 1.0000x reference)
import functools
import math

import jax
import jax.numpy as jnp
from jax import lax
from jax.experimental import pallas as pl
from jax.experimental.pallas import tpu as pltpu
from jax.experimental.pallas import tpu_sc as plsc

_MARGIN = 0.1
_SCALE = 64.0
_K2 = _SCALE * math.log2(math.e)
_COS_M = math.cos(_MARGIN)
_SIN_M = math.sin(_MARGIN)

_RB = 4096
_SC_SLABS = 1664


def _sumexp_tc_body(ct_ref, out_ref, acc_ref, *, B, C_tc, n_blocks):
    i = pl.program_id(0)

    @pl.when(i == 0)
    def _init():
        acc_ref[...] = jnp.zeros_like(acc_ref)

    c = ct_ref[...]

    n_full = C_tc // _RB

    @pl.when(i < n_full)
    def _main():
        e = jnp.exp2(c * _K2 - _K2)
        acc_ref[...] += jnp.sum(e.reshape(_RB // 8, 8, B), axis=0)

    @pl.when(i >= n_full)
    def _tail():
        row = jax.lax.broadcasted_iota(jnp.int32, (_RB, B), 0) + i * _RB
        e = jnp.where(row < C_tc, jnp.exp2(c * _K2 - _K2), 0.0)
        acc_ref[...] += jnp.sum(e.reshape(_RB // 8, 8, B), axis=0)

    @pl.when(i == n_blocks - 1)
    def _finish():
        out_ref[...] = jnp.sum(acc_ref[...], axis=0, keepdims=True)


def _gather_sc_body(
    tbl_ref, lab_ref, g_ref, t_ref, labv, idxv, slab, outv, accv, bufa, bufb,
    sem, sema, semb,
):
    B = lab_ref.shape[0]
    n_slabs = tbl_ref.shape[0]
    info = plsc.get_sparse_core_info()
    nc = info.num_cores
    n_w = nc * info.num_subcores
    wid = lax.axis_index("s") * nc + lax.axis_index("c")
    b_per_w = B // n_w
    base = wid * b_per_w
    cbase = (base // 128) * 128
    coff = base - cbase

    k_per_w = _SC_SLABS // n_w
    sbase = (n_slabs - _SC_SLABS) + wid * k_per_w
    pltpu.async_copy(tbl_ref.at[pl.ds(sbase, 1)], bufa, sema)
    pltpu.async_copy(tbl_ref.at[pl.ds(sbase + 1, 1)], bufb, semb)

    pltpu.sync_copy(lab_ref.at[pl.ds(base, b_per_w)], labv)
    for h in range(b_per_w // 16):
        lv = jnp.maximum(labv[pl.ds(h * 16, 16)], 0)
        idxv[pl.ds(h * 16, 16)] = lax.shift_right_logical(lv, 3)
    j16 = lax.broadcasted_iota(jnp.int32, (16,), 0)
    for h in range(b_per_w // 16):
        pltpu.async_copy(
            tbl_ref.at[idxv.at[pl.ds(h * 16, 16)], :, pl.ds(cbase, 128)],
            slab,
            sem,
        ).wait()
        lvh = jnp.maximum(labv[pl.ds(h * 16, 16)], 0)
        row = jnp.bitwise_and(lvh, 7)
        col = coff + h * 16 + j16
        vals = plsc.load_gather(slab, [j16, row, col])
        outv[pl.ds(h * 16, 16)] = vals
    pltpu.sync_copy(outv, g_ref.at[pl.ds(base, b_per_w)])

    zero = jnp.zeros((16,), jnp.float32)
    for k in range(B // 16):
        accv[pl.ds(k * 16, 16)] = zero

    def _reduce(buf):
        for c1 in range(8):
            for kk in range(8):
                off = c1 * 128 + kk * 16
                a = accv[pl.ds(off, 16)]
                for r in range(8):
                    x = buf[0, r, pl.ds(off, 16)]
                    a = a + jnp.exp(x * _SCALE - _SCALE)
                accv[pl.ds(off, 16)] = a

    def _pair(p, _):
        s = sbase + 2 * p
        pltpu.make_async_copy(tbl_ref.at[pl.ds(s, 1)], bufa, sema).wait()
        _reduce(bufa)

        @pl.when(p < k_per_w // 2 - 1)
        def _():
            pltpu.async_copy(tbl_ref.at[pl.ds(s + 2, 1)], bufa, sema)

        pltpu.make_async_copy(tbl_ref.at[pl.ds(s + 1, 1)], bufb, semb).wait()
        _reduce(bufb)

        @pl.when(p < k_per_w // 2 - 1)
        def _():
            pltpu.async_copy(tbl_ref.at[pl.ds(s + 3, 1)], bufb, semb)

        return 0

    lax.fori_loop(0, k_per_w // 2, _pair, 0)
    pltpu.sync_copy(accv, t_ref.at[pl.ds(wid * B, B)])


def _combine_tc_body(S_ref, T_ref, g_ref, lab_ref, out_ref, *, B):
    S = S_ref[...] + jnp.sum(T_ref[...], axis=0, keepdims=True)
    g = g_ref[...]
    lab = lab_ref[...]
    valid = lab >= 0
    o = _SCALE * g
    sin_t = jnp.sqrt(jnp.maximum(1.0 - g * g, 0.0))
    v = jnp.where(valid, _SCALE * (_COS_M * g - _SIN_M * sin_t), o)
    S_corr = S - jnp.exp(o - _SCALE) + jnp.exp(v - _SCALE)
    loss_i = _SCALE - v + jnp.log(S_corr)
    out_ref[...] = jnp.sum(loss_i, axis=1, keepdims=True) / B


def kernel(cosine, label):
    B, C = cosine.shape
    ct = cosine.T
    lab_i32 = label.astype(jnp.int32)
    C_tc = C - 8 * _SC_SLABS
    n_blocks = pl.cdiv(C_tc, _RB)

    tbl = ct.reshape(C // 8, 8, B)
    mesh = plsc.VectorSubcoreMesh(core_axis_name="c", subcore_axis_name="s")
    n_w = mesh.num_cores * mesh.num_subcores
    b_per_w = B // n_w
    g, tails = pl.kernel(
        _gather_sc_body,
        out_type=(
            jax.ShapeDtypeStruct((B,), jnp.float32),
            jax.ShapeDtypeStruct((n_w * B,), jnp.float32),
        ),
        mesh=mesh,
        compiler_params=pltpu.CompilerParams(needs_layout_passes=False),
        scratch_types=[
            pltpu.VMEM((b_per_w,), jnp.int32),
            pltpu.VMEM((b_per_w,), jnp.int32),
            pltpu.VMEM((16, 8, 128), jnp.float32),
            pltpu.VMEM((b_per_w,), jnp.float32),
            pltpu.VMEM((B,), jnp.float32),
            pltpu.VMEM((1, 8, B), jnp.float32),
            pltpu.VMEM((1, 8, B), jnp.float32),
            pltpu.SemaphoreType.DMA,
            pltpu.SemaphoreType.DMA,
            pltpu.SemaphoreType.DMA,
        ],
    )(tbl, lab_i32)

    S = pl.pallas_call(
        functools.partial(_sumexp_tc_body, B=B, C_tc=C_tc, n_blocks=n_blocks),
        grid=(n_blocks,),
        in_specs=[pl.BlockSpec((_RB, B), lambda i: (i, 0))],
        out_specs=pl.BlockSpec((1, B), lambda i: (0, 0)),
        out_shape=jax.ShapeDtypeStruct((1, B), jnp.float32),
        scratch_shapes=[pltpu.VMEM((8, B), jnp.float32)],
    )(ct)

    out = pl.pallas_call(
        functools.partial(_combine_tc_body, B=B),
        out_shape=jax.ShapeDtypeStruct((1, 1), jnp.float32),
    )(S, tails.reshape(n_w, B), g.reshape(1, B), lab_i32.reshape(1, B))
    return out[0, 0]


# speedup vs baseline: 1.0208x; 1.0208x over previous
import functools
import math

import jax
import jax.numpy as jnp
from jax import lax
from jax.experimental import pallas as pl
from jax.experimental.pallas import tpu as pltpu
from jax.experimental.pallas import tpu_sc as plsc

_MARGIN = 0.1
_SCALE = 64.0
_K2 = _SCALE * math.log2(math.e)
_COS_M = math.cos(_MARGIN)
_SIN_M = math.sin(_MARGIN)

_RB = 4096
_SC_SLABS = 1280


def _sumexp_tc_body(ct_ref, out_ref, acc_ref, *, B, C_tc, n_blocks):
    i = pl.program_id(0)

    @pl.when(i == 0)
    def _init():
        acc_ref[...] = jnp.zeros_like(acc_ref)

    c = ct_ref[...]

    n_full = C_tc // _RB

    @pl.when(i < n_full)
    def _main():
        e = jnp.exp2(c * _K2 - _K2)
        acc_ref[...] += jnp.sum(e.reshape(_RB // 8, 8, B), axis=0)

    @pl.when(i >= n_full)
    def _tail():
        row = jax.lax.broadcasted_iota(jnp.int32, (_RB, B), 0) + i * _RB
        e = jnp.where(row < C_tc, jnp.exp2(c * _K2 - _K2), 0.0)
        acc_ref[...] += jnp.sum(e.reshape(_RB // 8, 8, B), axis=0)

    @pl.when(i == n_blocks - 1)
    def _finish():
        out_ref[...] = jnp.sum(acc_ref[...], axis=0, keepdims=True)


def _gather_sc_body(
    tbl_ref, lab_ref, g_ref, t_ref, labv, idxv, slab, outv, accv, bufa, bufb,
    sem, sema, semb,
):
    B = lab_ref.shape[0]
    n_slabs = tbl_ref.shape[0]
    info = plsc.get_sparse_core_info()
    nc = info.num_cores
    n_w = nc * info.num_subcores
    wid = lax.axis_index("s") * nc + lax.axis_index("c")
    b_per_w = B // n_w
    base = wid * b_per_w
    cbase = (base // 128) * 128
    coff = base - cbase

    k_per_w = _SC_SLABS // n_w
    sbase = (n_slabs - _SC_SLABS) + wid * k_per_w
    pltpu.async_copy(tbl_ref.at[pl.ds(sbase, 1)], bufa, sema)
    pltpu.async_copy(tbl_ref.at[pl.ds(sbase + 1, 1)], bufb, semb)

    pltpu.sync_copy(lab_ref.at[pl.ds(base, b_per_w)], labv)
    for h in range(b_per_w // 16):
        lv = jnp.maximum(labv[pl.ds(h * 16, 16)], 0)
        idxv[pl.ds(h * 16, 16)] = lax.shift_right_logical(lv, 3)
    j16 = lax.broadcasted_iota(jnp.int32, (16,), 0)
    for h in range(b_per_w // 16):
        pltpu.async_copy(
            tbl_ref.at[idxv.at[pl.ds(h * 16, 16)], :, pl.ds(cbase, 128)],
            slab,
            sem,
        ).wait()
        lvh = jnp.maximum(labv[pl.ds(h * 16, 16)], 0)
        row = jnp.bitwise_and(lvh, 7)
        col = coff + h * 16 + j16
        vals = plsc.load_gather(slab, [j16, row, col])
        outv[pl.ds(h * 16, 16)] = vals
    pltpu.sync_copy(outv, g_ref.at[pl.ds(base, b_per_w)])

    zero = jnp.zeros((16,), jnp.float32)
    for k in range(B // 16):
        accv[pl.ds(k * 16, 16)] = zero

    def _reduce(buf):
        for c1 in range(8):
            for kk in range(8):
                off = c1 * 128 + kk * 16
                a = accv[pl.ds(off, 16)]
                for r in range(8):
                    x = buf[0, r, pl.ds(off, 16)]
                    a = a + jnp.exp(x * _SCALE - _SCALE)
                accv[pl.ds(off, 16)] = a

    def _pair(p, _):
        s = sbase + 2 * p
        pltpu.make_async_copy(tbl_ref.at[pl.ds(s, 1)], bufa, sema).wait()
        _reduce(bufa)

        @pl.when(p < k_per_w // 2 - 1)
        def _():
            pltpu.async_copy(tbl_ref.at[pl.ds(s + 2, 1)], bufa, sema)

        pltpu.make_async_copy(tbl_ref.at[pl.ds(s + 1, 1)], bufb, semb).wait()
        _reduce(bufb)

        @pl.when(p < k_per_w // 2 - 1)
        def _():
            pltpu.async_copy(tbl_ref.at[pl.ds(s + 3, 1)], bufb, semb)

        return 0

    lax.fori_loop(0, k_per_w // 2, _pair, 0)
    pltpu.sync_copy(accv, t_ref.at[pl.ds(wid * B, B)])


def _combine_tc_body(S_ref, T_ref, g_ref, lab_ref, out_ref, *, B):
    S = S_ref[...] + jnp.sum(T_ref[...], axis=0, keepdims=True)
    g = g_ref[...]
    lab = lab_ref[...]
    valid = lab >= 0
    o = _SCALE * g
    sin_t = jnp.sqrt(jnp.maximum(1.0 - g * g, 0.0))
    v = jnp.where(valid, _SCALE * (_COS_M * g - _SIN_M * sin_t), o)
    S_corr = S - jnp.exp(o - _SCALE) + jnp.exp(v - _SCALE)
    loss_i = _SCALE - v + jnp.log(S_corr)
    out_ref[...] = jnp.sum(loss_i, axis=1, keepdims=True) / B


def kernel(cosine, label):
    B, C = cosine.shape
    ct = cosine.T
    lab_i32 = label.astype(jnp.int32)
    C_tc = C - 8 * _SC_SLABS
    n_blocks = pl.cdiv(C_tc, _RB)

    tbl = ct.reshape(C // 8, 8, B)
    mesh = plsc.VectorSubcoreMesh(core_axis_name="c", subcore_axis_name="s")
    n_w = mesh.num_cores * mesh.num_subcores
    b_per_w = B // n_w
    g, tails = pl.kernel(
        _gather_sc_body,
        out_type=(
            jax.ShapeDtypeStruct((B,), jnp.float32),
            jax.ShapeDtypeStruct((n_w * B,), jnp.float32),
        ),
        mesh=mesh,
        compiler_params=pltpu.CompilerParams(needs_layout_passes=False),
        scratch_types=[
            pltpu.VMEM((b_per_w,), jnp.int32),
            pltpu.VMEM((b_per_w,), jnp.int32),
            pltpu.VMEM((16, 8, 128), jnp.float32),
            pltpu.VMEM((b_per_w,), jnp.float32),
            pltpu.VMEM((B,), jnp.float32),
            pltpu.VMEM((1, 8, B), jnp.float32),
            pltpu.VMEM((1, 8, B), jnp.float32),
            pltpu.SemaphoreType.DMA,
            pltpu.SemaphoreType.DMA,
            pltpu.SemaphoreType.DMA,
        ],
    )(tbl, lab_i32)

    S = pl.pallas_call(
        functools.partial(_sumexp_tc_body, B=B, C_tc=C_tc, n_blocks=n_blocks),
        grid=(n_blocks,),
        in_specs=[pl.BlockSpec((_RB, B), lambda i: (i, 0))],
        out_specs=pl.BlockSpec((1, B), lambda i: (0, 0)),
        out_shape=jax.ShapeDtypeStruct((1, B), jnp.float32),
        scratch_shapes=[pltpu.VMEM((8, B), jnp.float32)],
    )(ct)

    out = pl.pallas_call(
        functools.partial(_combine_tc_body, B=B),
        out_shape=jax.ShapeDtypeStruct((1, 1), jnp.float32),
    )(S, tails.reshape(n_w, B), g.reshape(1, B), lab_i32.reshape(1, B))
    return out[0, 0]
